# baseline (speedup 1.0000x reference)
import functools

import jax
import jax.numpy as jnp
from jax import lax
from jax.experimental import pallas as pl
from jax.experimental.pallas import tpu as pltpu
from jax.experimental.pallas import tpu_sc as plsc

_N = 26
_K = 100000
_B = 16384
_NC = 2
_NS = 16
_LANES = 16
_CH = 4096
_NCH = _B // _CH
_CK = 8192
_NBK = -(-_K // _CK)


def _sc_gather_body(p_hbm, idx_hbm, out_hbm, row_v, i0, i1, i2, i3, g0, g1,
                    sem_row, sem_idx, sem_o0, sem_o1):
    wid = lax.axis_index("s") * _NC + lax.axis_index("c")

    @pl.when(wid < _N)
    def _():
        idx_bufs = (i0, i1, i2, i3)
        g_bufs = (g0, g1)
        osems = (sem_o0, sem_o1)

        row_cp = pltpu.async_copy(p_hbm.at[wid], row_v, sem_row)
        icps = [
            pltpu.async_copy(
                idx_hbm.at[wid, pl.ds(c * _CH, _CH)], idx_bufs[c], sem_idx)
            for c in range(_NCH)
        ]
        row_cp.wait()

        ocps = [None, None]
        for c in range(_NCH):
            b = c % 2
            icps[c].wait()
            if ocps[b] is not None:
                ocps[b].wait()

            def _gather(ib, gb):
                @plsc.parallel_loop(0, _CH, _LANES, unroll=8)
                def _g(i):
                    sl = pl.ds(i, _LANES)
                    gb[sl] = plsc.load_gather(row_v, [ib[sl]])

            _gather(idx_bufs[c], g_bufs[b])
            ocps[b] = pltpu.async_copy(
                g_bufs[b], out_hbm.at[wid, pl.ds(c * _CH, _CH)], osems[b])
        ocps[0].wait()
        ocps[1].wait()


_sc_gather = functools.partial(
    pl.kernel,
    out_type=jax.ShapeDtypeStruct((_N, _B), jnp.float32),
    mesh=plsc.VectorSubcoreMesh(core_axis_name="c", subcore_axis_name="s"),
    compiler_params=pltpu.CompilerParams(needs_layout_passes=False),
    scratch_types=[
        pltpu.VMEM((_K,), jnp.float32),
        pltpu.VMEM((_CH,), jnp.int32),
        pltpu.VMEM((_CH,), jnp.int32),
        pltpu.VMEM((_CH,), jnp.int32),
        pltpu.VMEM((_CH,), jnp.int32),
        pltpu.VMEM((_CH,), jnp.float32),
        pltpu.VMEM((_CH,), jnp.float32),
        pltpu.SemaphoreType.DMA,
        pltpu.SemaphoreType.DMA,
        pltpu.SemaphoreType.DMA,
        pltpu.SemaphoreType.DMA,
    ],
)(_sc_gather_body)


def _tc_sum_body(p_ref, s_ref):
    i = pl.program_id(0)

    @pl.when(i == 0)
    def _():
        s_ref[...] = jnp.zeros_like(s_ref)

    x = p_ref[...]
    col = i * _CK + lax.broadcasted_iota(jnp.int32, x.shape, 1)
    sig = jnp.where(col < _K, jax.nn.sigmoid(x), 0.0)
    part = jnp.sum(sig, axis=1, keepdims=True)
    s_ref[...] += jnp.broadcast_to(part, s_ref.shape)


_TB = 2048


def _tc_tin_body(d_ref, o_ref):
    o_ref[...] = d_ref[...].T


def _tc_finish_body(g_ref, s_ref, o_ref):
    s = s_ref[:, 0:1]
    o_ref[...] = jnp.log(jax.nn.sigmoid(g_ref[...]) / s).T


def kernel(data, p_aux):
    idx_t = pl.pallas_call(
        _tc_tin_body,
        grid=(_B // _TB,),
        in_specs=[pl.BlockSpec((_TB, _N), lambda i: (i, 0))],
        out_specs=pl.BlockSpec((_N, _TB), lambda i: (0, i)),
        out_shape=jax.ShapeDtypeStruct((_N, _B), jnp.int32),
    )(data)
    s = pl.pallas_call(
        _tc_sum_body,
        grid=(_NBK,),
        in_specs=[pl.BlockSpec((_N, _CK), lambda i: (0, i))],
        out_specs=pl.BlockSpec((_N, 128), lambda i: (0, 0)),
        out_shape=jax.ShapeDtypeStruct((_N, 128), jnp.float32),
    )(p_aux)
    g_t = _sc_gather(p_aux, idx_t)
    out = pl.pallas_call(
        _tc_finish_body,
        grid=(_B // _TB,),
        in_specs=[
            pl.BlockSpec((_N, _TB), lambda i: (0, i)),
            pl.BlockSpec((_N, 128), lambda i: (0, 0)),
        ],
        out_specs=pl.BlockSpec((_TB, _N), lambda i: (i, 0)),
        out_shape=jax.ShapeDtypeStruct((_B, _N), jnp.float32),
    )(g_t, s)
    return out


# speedup vs baseline: 1.4069x; 1.4069x over previous
import functools

import jax
import jax.numpy as jnp
from jax import lax
from jax.experimental import pallas as pl
from jax.experimental.pallas import tpu as pltpu
from jax.experimental.pallas import tpu_sc as plsc

_N = 26
_K = 100000
_B = 16384
_NC = 2
_LANES = 16
_CH = 4096
_NCH = _B // _CH
_CK = 8192
_NBK = -(-_K // _CK)

_LN2 = 0.6931471805599453
_SQRT2 = 1.4142135623730951


def _ln(x):
    bits = lax.bitcast_convert_type(x, jnp.int32)
    e = lax.shift_right_logical(bits, 23) - 127
    m = lax.bitcast_convert_type(
        (bits & 0x7FFFFF) | 0x3F800000, jnp.float32)
    big = m > _SQRT2
    m = jnp.where(big, m * 0.5, m)
    e = (e + big.astype(jnp.int32)).astype(jnp.float32)
    t = m - 1.0
    p = 1.0 / 7.0
    for c in (-1.0 / 6.0, 1.0 / 5.0, -1.0 / 4.0, 1.0 / 3.0, -1.0 / 2.0, 1.0):
        p = p * t + c
    return e * _LN2 + t * p


def _sc_body(p_hbm, idx_hbm, ls_hbm, out_hbm, row_v, ls_v, i0, i1, i2, i3,
             g0, g1, sem_row, sem_ls, sem_idx, sem_o0, sem_o1):
    wid = lax.axis_index("s") * _NC + lax.axis_index("c")

    @pl.when(wid < _N)
    def _():
        idx_bufs = (i0, i1, i2, i3)
        g_bufs = (g0, g1)
        osems = (sem_o0, sem_o1)

        row_cp = pltpu.async_copy(p_hbm.at[wid], row_v, sem_row)
        ls_cp = pltpu.async_copy(ls_hbm.at[wid, pl.ds(0, _LANES)], ls_v, sem_ls)
        icps = [
            pltpu.async_copy(
                idx_hbm.at[wid, pl.ds(c * _CH, _CH)], idx_bufs[c], sem_idx)
            for c in range(_NCH)
        ]
        row_cp.wait()
        ls_cp.wait()
        logs = ls_v[...]

        ocps = [None, None]
        for c in range(_NCH):
            b = c % 2
            icps[c].wait()
            if ocps[b] is not None:
                ocps[b].wait()

            def _gather(ib, gb):
                @plsc.parallel_loop(0, _CH, _LANES, unroll=4)
                def _g(i):
                    sl = pl.ds(i, _LANES)
                    x = plsc.load_gather(row_v, [ib[sl]])
                    sig = 1.0 / (1.0 + jnp.exp(-x))
                    gb[sl] = _ln(sig) - logs

            _gather(idx_bufs[c], g_bufs[b])
            ocps[b] = pltpu.async_copy(
                g_bufs[b], out_hbm.at[wid, pl.ds(c * _CH, _CH)], osems[b])
        ocps[0].wait()
        ocps[1].wait()


@functools.lru_cache(maxsize=None)
def _sc_call():
  return functools.partial(
    pl.kernel,
    out_type=jax.ShapeDtypeStruct((_N, _B), jnp.float32),
    mesh=plsc.VectorSubcoreMesh(core_axis_name="c", subcore_axis_name="s"),
    compiler_params=pltpu.CompilerParams(needs_layout_passes=False),
    scratch_types=[
        pltpu.VMEM((_K,), jnp.float32),
        pltpu.VMEM((_LANES,), jnp.float32),
        pltpu.VMEM((_CH,), jnp.int32),
        pltpu.VMEM((_CH,), jnp.int32),
        pltpu.VMEM((_CH,), jnp.int32),
        pltpu.VMEM((_CH,), jnp.int32),
        pltpu.VMEM((_CH,), jnp.float32),
        pltpu.VMEM((_CH,), jnp.float32),
        pltpu.SemaphoreType.DMA,
        pltpu.SemaphoreType.DMA,
        pltpu.SemaphoreType.DMA,
        pltpu.SemaphoreType.DMA,
        pltpu.SemaphoreType.DMA,
    ],
  )(_sc_body)


def _tc_sum_body(p_ref, s_ref):
    i = pl.program_id(0)

    @pl.when(i == 0)
    def _():
        s_ref[...] = jnp.zeros_like(s_ref)

    x = p_ref[...]
    col = i * _CK + lax.broadcasted_iota(jnp.int32, x.shape, 1)
    sig = jnp.where(col < _K, jax.nn.sigmoid(x), 0.0)
    part = jnp.sum(sig, axis=1, keepdims=True)
    s_ref[...] += jnp.broadcast_to(part, s_ref.shape)

    @pl.when(i == _NBK - 1)
    def _():
        s_ref[...] = jnp.log(s_ref[...])


def kernel(data, p_aux):
    idx_t = data.T
    logs = pl.pallas_call(
        _tc_sum_body,
        grid=(_NBK,),
        in_specs=[pl.BlockSpec((_N, _CK), lambda i: (0, i))],
        out_specs=pl.BlockSpec((_N, 128), lambda i: (0, 0)),
        out_shape=jax.ShapeDtypeStruct((_N, 128), jnp.float32),
    )(p_aux)
    out_t = _sc_call()(p_aux, idx_t, logs)
    return out_t.T


# speedup vs baseline: 1.7877x; 1.2707x over previous
import functools

import jax
import jax.numpy as jnp
from jax import lax
from jax.experimental import pallas as pl
from jax.experimental.pallas import tpu as pltpu
from jax.experimental.pallas import tpu_sc as plsc

_N = 26
_K = 100000
_B = 16384
_NC = 2
_LANES = 16
_CH = 4096
_NCH = _B // _CH
_CK = 25088
_NBK = -(-_K // _CK)
_FB = 4096


def _sc_gather_body(p_hbm, idx_hbm, out_hbm, row_v, i0, i1, i2, i3, g0, g1,
                    sem_row, sem_idx, sem_o0, sem_o1):
    wid = lax.axis_index("s") * _NC + lax.axis_index("c")

    @pl.when(wid < _N)
    def _():
        idx_bufs = (i0, i1, i2, i3)
        g_bufs = (g0, g1)
        osems = (sem_o0, sem_o1)

        row_cp = pltpu.async_copy(p_hbm.at[wid], row_v, sem_row)
        icps = [
            pltpu.async_copy(
                idx_hbm.at[wid, pl.ds(c * _CH, _CH)], idx_bufs[c], sem_idx)
            for c in range(_NCH)
        ]
        row_cp.wait()

        ocps = [None, None]
        for c in range(_NCH):
            b = c % 2
            icps[c].wait()
            if ocps[b] is not None:
                ocps[b].wait()

            def _gather(ib, gb):
                @plsc.parallel_loop(0, _CH, _LANES, unroll=8)
                def _g(i):
                    sl = pl.ds(i, _LANES)
                    gb[sl] = plsc.load_gather(row_v, [ib[sl]])

            _gather(idx_bufs[c], g_bufs[b])
            ocps[b] = pltpu.async_copy(
                g_bufs[b], out_hbm.at[wid, pl.ds(c * _CH, _CH)], osems[b])
        ocps[0].wait()
        ocps[1].wait()


@functools.lru_cache(maxsize=None)
def _sc_gather():
  return functools.partial(
    pl.kernel,
    out_type=jax.ShapeDtypeStruct((_N, _B), jnp.float32),
    mesh=plsc.VectorSubcoreMesh(core_axis_name="c", subcore_axis_name="s"),
    compiler_params=pltpu.CompilerParams(needs_layout_passes=False),
    scratch_types=[
        pltpu.VMEM((_K,), jnp.float32),
        pltpu.VMEM((_CH,), jnp.int32),
        pltpu.VMEM((_CH,), jnp.int32),
        pltpu.VMEM((_CH,), jnp.int32),
        pltpu.VMEM((_CH,), jnp.int32),
        pltpu.VMEM((_CH,), jnp.float32),
        pltpu.VMEM((_CH,), jnp.float32),
        pltpu.SemaphoreType.DMA,
        pltpu.SemaphoreType.DMA,
        pltpu.SemaphoreType.DMA,
        pltpu.SemaphoreType.DMA,
    ],
  )(_sc_gather_body)


def _tc_sum_body(p_ref, s_ref):
    i = pl.program_id(0)

    @pl.when(i == 0)
    def _():
        s_ref[...] = jnp.zeros_like(s_ref)

    x = p_ref[...]
    col = i * _CK + lax.broadcasted_iota(jnp.int32, x.shape, 1)
    sig = jnp.where(col < _K, jax.nn.sigmoid(x), 0.0)
    part = jnp.sum(sig, axis=1, keepdims=True)
    s_ref[...] += jnp.broadcast_to(part, s_ref.shape)


def _tc_finish_body(g_ref, s_ref, o_ref):
    s = s_ref[:, 0:1]
    o_ref[...] = jnp.log(jax.nn.sigmoid(g_ref[...]) / s)


def kernel(data, p_aux):
    idx_t = data.T
    s = pl.pallas_call(
        _tc_sum_body,
        grid=(_NBK,),
        in_specs=[pl.BlockSpec((_N, _CK), lambda i: (0, i))],
        out_specs=pl.BlockSpec((_N, 128), lambda i: (0, 0)),
        out_shape=jax.ShapeDtypeStruct((_N, 128), jnp.float32),
    )(p_aux)
    g_t = _sc_gather()(p_aux, idx_t)
    out_t = pl.pallas_call(
        _tc_finish_body,
        grid=(_B // _FB,),
        in_specs=[
            pl.BlockSpec((_N, _FB), lambda i: (0, i)),
            pl.BlockSpec((_N, 128), lambda i: (0, 0)),
        ],
        out_specs=pl.BlockSpec((_N, _FB), lambda i: (0, i)),
        out_shape=jax.ShapeDtypeStruct((_N, _B), jnp.float32),
    )(g_t, s)
    return out_t.T
